# initial kernel scaffold (unmeasured)
import jax
import jax.numpy as jnp
from jax import lax
from jax.experimental import pallas as pl
from jax.experimental.pallas import tpu as pltpu

_NEG = -1e30


def _cast_bf16(x):
    n = x.shape[0]
    blk = 16
    rest = x.shape[1:]

    def body(x_ref, o_ref):
        o_ref[...] = x_ref[...].astype(jnp.bfloat16)

    return pl.pallas_call(
        body,
        grid=(n // blk,),
        in_specs=[pl.BlockSpec((blk,) + rest, lambda i: (i, 0, 0, 0))],
        out_specs=pl.BlockSpec((blk,) + rest, lambda i: (i, 0, 0, 0)),
        out_shape=jax.ShapeDtypeStruct(x.shape, jnp.bfloat16),
    )(x)


def _local_paged_attn(loc, bt, lens, Q, K, V):
    B, _, H, D = Q.shape
    NP, BS = K.shape[0], K.shape[1]
    NB = bt.shape[1]
    scale = D ** -0.5

    def body(loc_ref, bt_ref, lens_ref, q_ref, k_ref, v_ref, o_ref, m_ref, l_ref):
        i = pl.program_id(0)
        j = pl.program_id(1)

        @pl.when(j == 0)
        def _():
            o_ref[...] = jnp.zeros_like(o_ref)
            m_ref[...] = jnp.full_like(m_ref, _NEG)
            l_ref[...] = jnp.zeros_like(l_ref)

        page = bt_ref[i, j]
        lo = loc_ref[0]
        valid = (j < lens_ref[i]) & (page >= lo) & (page < lo + NP)

        q = q_ref[0, 0].astype(jnp.bfloat16)
        k = k_ref[0]
        s = lax.dot_general(
            q, k, (((1,), (2,)), ((0,), (1,))),
            preferred_element_type=jnp.float32,
        ) * scale
        s = jnp.where(valid, s, _NEG)
        m_prev = m_ref[0, :, 0:1]
        l_prev = l_ref[0, :, 0:1]
        m_cur = jnp.max(s, axis=1, keepdims=True)
        m_new = jnp.maximum(m_prev, m_cur)
        alpha = jnp.exp(m_prev - m_new)
        p = jnp.exp(s - m_new)
        p = jnp.where(valid, p, 0.0)
        l_new = l_prev * alpha + jnp.sum(p, axis=1, keepdims=True)
        pv = lax.dot_general(
            p.astype(jnp.bfloat16), v_ref[0],
            (((1,), (0,)), ((0,), (1,))),
            preferred_element_type=jnp.float32,
        )
        o_ref[0] = o_ref[0] * alpha + pv
        m_ref[0] = jnp.broadcast_to(m_new, (H, D))
        l_ref[0] = jnp.broadcast_to(l_new, (H, D))

    def kv_map(i, j, loc, bt, lens):
        return (jnp.clip(bt[i, j] - loc[0], 0, NP - 1), 0, 0, 0)

    def out_map(i, j, loc, bt, lens):
        return (i, 0, 0)

    grid_spec = pltpu.PrefetchScalarGridSpec(
        num_scalar_prefetch=3,
        grid=(B, NB),
        in_specs=[
            pl.BlockSpec((1, 1, H, D), lambda i, j, loc, bt, lens: (i, 0, 0, 0)),
            pl.BlockSpec((1, BS, H, D), kv_map),
            pl.BlockSpec((1, BS, H, D), kv_map),
        ],
        out_specs=[
            pl.BlockSpec((1, H, D), out_map),
            pl.BlockSpec((1, H, D), out_map),
            pl.BlockSpec((1, H, D), out_map),
        ],
    )
    return pl.pallas_call(
        body,
        grid_spec=grid_spec,
        out_shape=[jax.ShapeDtypeStruct((B, H, D), jnp.float32)] * 3,
        compiler_params=pltpu.CompilerParams(
            dimension_semantics=("arbitrary", "arbitrary"),
        ),
    )(loc, bt, lens, Q, K, V)


def _cross_x_combine(o, m, l):
    B, H, D = o.shape

    def body(o_ref, m_ref, l_ref, out_ref, ob, mb, lb, ssem, rsem):
        my_x = lax.axis_index("x")
        my_y = lax.axis_index("y")
        peer = (1 - my_x, my_y)
        rdmas = []
        for t, (src, dst) in enumerate(((o_ref, ob), (m_ref, mb), (l_ref, lb))):
            r = pltpu.make_async_remote_copy(
                src_ref=src,
                dst_ref=dst,
                send_sem=ssem.at[t],
                recv_sem=rsem.at[t],
                device_id=peer,
                device_id_type=pl.DeviceIdType.MESH,
            )
            r.start()
            rdmas.append(r)
        for r in rdmas:
            r.wait()
        m0 = m_ref[...]
        m1 = mb[...]
        mn = jnp.maximum(m0, m1)
        a0 = jnp.exp(m0 - mn)
        a1 = jnp.exp(m1 - mn)
        den = l_ref[...] * a0 + lb[...] * a1
        num = o_ref[...] * a0 + ob[...] * a1
        out_ref[...] = (num / den).reshape(B, 1, H, D)

    return pl.pallas_call(
        body,
        out_shape=jax.ShapeDtypeStruct((B, 1, H, D), jnp.float32),
        in_specs=[pl.BlockSpec(memory_space=pltpu.VMEM)] * 3,
        out_specs=pl.BlockSpec(memory_space=pltpu.VMEM),
        scratch_shapes=[
            pltpu.VMEM((B, H, D), jnp.float32),
            pltpu.VMEM((B, H, D), jnp.float32),
            pltpu.VMEM((B, H, D), jnp.float32),
            pltpu.SemaphoreType.DMA((3,)),
            pltpu.SemaphoreType.DMA((3,)),
        ],
        compiler_params=pltpu.CompilerParams(collective_id=0),
    )(o, m, l)


def kernel(Q, K, V, bt, lens):
    NP = K.shape[0]
    loc = jnp.reshape(lax.axis_index("x").astype(jnp.int32) * NP, (1,))
    K16 = _cast_bf16(K)
    V16 = _cast_bf16(V)
    o, m, l = _local_paged_attn(loc, bt, lens, Q, K16, V16)
    return _cross_x_combine(o, m, l)


# baseline (device time: 10726879 ns/iter reference)
import jax
import jax.numpy as jnp
from jax import lax
from jax.experimental import pallas as pl
from jax.experimental.pallas import tpu as pltpu

_NEG = -1e30


def _cast_bf16(x):
    n = x.shape[0]
    blk = 16
    rest = x.shape[1:]

    def body(x_ref, o_ref):
        o_ref[...] = x_ref[...].astype(jnp.bfloat16)

    return pl.pallas_call(
        body,
        grid=(n // blk,),
        in_specs=[pl.BlockSpec((blk,) + rest, lambda i: (i, 0, 0, 0))],
        out_specs=pl.BlockSpec((blk,) + rest, lambda i: (i, 0, 0, 0)),
        out_shape=jax.ShapeDtypeStruct(x.shape, jnp.bfloat16),
    )(x)


def _local_paged_attn(loc, bt, lens, Q, K, V):
    B, _, H, D = Q.shape
    NP, BS = K.shape[0], K.shape[1]
    NB = bt.shape[1]
    scale = D ** -0.5

    def body(loc_ref, bt_ref, lens_ref, q_ref, k_ref, v_ref, o_ref, m_ref, l_ref):
        i = pl.program_id(0)
        j = pl.program_id(1)

        @pl.when(j == 0)
        def _():
            o_ref[...] = jnp.zeros_like(o_ref)
            m_ref[...] = jnp.full_like(m_ref, _NEG)
            l_ref[...] = jnp.zeros_like(l_ref)

        page = bt_ref[i, j]
        lo = loc_ref[0]
        valid = (j < lens_ref[i]) & (page >= lo) & (page < lo + NP)

        q = q_ref[0, 0].astype(jnp.bfloat16)
        k = k_ref[0]
        s = lax.dot_general(
            q, k, (((1,), (2,)), ((0,), (1,))),
            preferred_element_type=jnp.float32,
        ) * scale
        s = jnp.where(valid, s, _NEG)
        m_prev = m_ref[0, :, 0:1]
        l_prev = l_ref[0, :, 0:1]
        m_cur = jnp.max(s, axis=1, keepdims=True)
        m_new = jnp.maximum(m_prev, m_cur)
        alpha = jnp.exp(m_prev - m_new)
        p = jnp.exp(s - m_new)
        p = jnp.where(valid, p, 0.0)
        l_new = l_prev * alpha + jnp.sum(p, axis=1, keepdims=True)
        pv = lax.dot_general(
            p.astype(jnp.bfloat16), v_ref[0],
            (((1,), (0,)), ((0,), (1,))),
            preferred_element_type=jnp.float32,
        )
        o_ref[0] = o_ref[0] * alpha + pv
        m_ref[0] = jnp.broadcast_to(m_new, (H, D))
        l_ref[0] = jnp.broadcast_to(l_new, (H, D))

    def kv_map(i, j, loc, bt, lens):
        return (jnp.clip(bt[i, j] - loc[0], 0, NP - 1), 0, 0, 0)

    def out_map(i, j, loc, bt, lens):
        return (i, 0, 0)

    grid_spec = pltpu.PrefetchScalarGridSpec(
        num_scalar_prefetch=3,
        grid=(B, NB),
        in_specs=[
            pl.BlockSpec((1, 1, H, D), lambda i, j, loc, bt, lens: (i, 0, 0, 0)),
            pl.BlockSpec((1, BS, H, D), kv_map),
            pl.BlockSpec((1, BS, H, D), kv_map),
        ],
        out_specs=[
            pl.BlockSpec((1, H, D), out_map),
            pl.BlockSpec((1, H, D), out_map),
            pl.BlockSpec((1, H, D), out_map),
        ],
    )
    return pl.pallas_call(
        body,
        grid_spec=grid_spec,
        out_shape=[jax.ShapeDtypeStruct((B, H, D), jnp.float32)] * 3,
        compiler_params=pltpu.CompilerParams(
            dimension_semantics=("arbitrary", "arbitrary"),
        ),
    )(loc, bt, lens, Q, K, V)


def _cross_x_combine(o, m, l):
    B, H, D = o.shape

    def body(o_ref, m_ref, l_ref, out_ref, ob, mb, lb, ssem, rsem):
        my_x = lax.axis_index("x")
        my_y = lax.axis_index("y")
        peer = (1 - my_x, my_y)
        rdmas = []
        for t, (src, dst) in enumerate(((o_ref, ob), (m_ref, mb), (l_ref, lb))):
            r = pltpu.make_async_remote_copy(
                src_ref=src,
                dst_ref=dst,
                send_sem=ssem.at[t],
                recv_sem=rsem.at[t],
                device_id=peer,
                device_id_type=pl.DeviceIdType.MESH,
            )
            r.start()
            rdmas.append(r)
        for r in rdmas:
            r.wait()
        m0 = m_ref[...]
        m1 = mb[...]
        mn = jnp.maximum(m0, m1)
        a0 = jnp.exp(m0 - mn)
        a1 = jnp.exp(m1 - mn)
        den = l_ref[...] * a0 + lb[...] * a1
        num = o_ref[...] * a0 + ob[...] * a1
        out_ref[...] = (num / den).reshape(B, 1, H, D)

    return pl.pallas_call(
        body,
        out_shape=jax.ShapeDtypeStruct((B, 1, H, D), jnp.float32),
        in_specs=[pl.BlockSpec(memory_space=pltpu.VMEM)] * 3,
        out_specs=pl.BlockSpec(memory_space=pltpu.VMEM),
        scratch_shapes=[
            pltpu.VMEM((B, H, D), jnp.float32),
            pltpu.VMEM((B, H, D), jnp.float32),
            pltpu.VMEM((B, H, D), jnp.float32),
            pltpu.SemaphoreType.DMA((3,)),
            pltpu.SemaphoreType.DMA((3,)),
        ],
    )(o, m, l)


def kernel(Q, K, V, bt, lens):
    NP = K.shape[0]
    loc = jnp.reshape(lax.axis_index("x").astype(jnp.int32) * NP, (1,))
    K16 = _cast_bf16(K)
    V16 = _cast_bf16(V)
    o, m, l = _local_paged_attn(loc, bt, lens, Q, K16, V16)
    return _cross_x_combine(o, m, l)


# device time: 7149062 ns/iter; 1.5005x vs baseline; 1.5005x over previous
import jax
import jax.numpy as jnp
from jax import lax
from jax.experimental import pallas as pl
from jax.experimental.pallas import tpu as pltpu

_NEG = -1e30


def _cast_bf16(x):
    n = x.shape[0]
    blk = 16
    rest = x.shape[1:]

    def body(x_ref, o_ref):
        o_ref[...] = x_ref[...].astype(jnp.bfloat16)

    return pl.pallas_call(
        body,
        grid=(n // blk,),
        in_specs=[pl.BlockSpec((blk,) + rest, lambda i: (i, 0, 0, 0))],
        out_specs=pl.BlockSpec((blk,) + rest, lambda i: (i, 0, 0, 0)),
        out_shape=jax.ShapeDtypeStruct(x.shape, jnp.bfloat16),
    )(x)


_CH = 8


def _local_paged_attn(loc, bt, lens, Q, K, V):
    B, _, H, D = Q.shape
    NP, BS = K.shape[0], K.shape[1]
    NB = bt.shape[1]
    CH = _CH
    scale = D ** -0.5

    def body(loc_ref, bt_ref, lens_ref, q_ref, *rest):
        k_refs = rest[:CH]
        v_refs = rest[CH:2 * CH]
        o_ref, m_ref, l_ref = rest[2 * CH:2 * CH + 3]
        i = pl.program_id(0)
        j = pl.program_id(1)

        @pl.when(j == 0)
        def _():
            o_ref[...] = jnp.zeros_like(o_ref)
            m_ref[...] = jnp.full_like(m_ref, _NEG)
            l_ref[...] = jnp.zeros_like(l_ref)

        lo = loc_ref[0]
        q = q_ref[0, 0].astype(jnp.bfloat16)

        valids = []
        s_parts = []
        for t in range(CH):
            jj = j * CH + t
            page = bt_ref[i, jj]
            valid = (jj < lens_ref[i]) & (page >= lo) & (page < lo + NP)
            s_t = lax.dot_general(
                q, k_refs[t][0], (((1,), (2,)), ((0,), (1,))),
                preferred_element_type=jnp.float32,
            ) * scale
            s_parts.append(jnp.where(valid, s_t, _NEG))
            valids.append(valid)
        s = jnp.concatenate(s_parts, axis=1)

        m_prev = m_ref[0, :, 0:1]
        l_prev = l_ref[0, :, 0:1]
        m_cur = jnp.max(s, axis=1, keepdims=True)
        m_new = jnp.maximum(m_prev, m_cur)
        alpha = jnp.exp(m_prev - m_new)
        p = jnp.exp(s - m_new)

        l_acc = l_prev * alpha
        pv_acc = o_ref[0] * alpha
        for t in range(CH):
            p_t = jnp.where(valids[t], p[:, t * BS:(t + 1) * BS], 0.0)
            l_acc = l_acc + jnp.sum(p_t, axis=1, keepdims=True)
            pv_acc = pv_acc + lax.dot_general(
                p_t.astype(jnp.bfloat16), v_refs[t][0],
                (((1,), (0,)), ((0,), (1,))),
                preferred_element_type=jnp.float32,
            )
        o_ref[0] = pv_acc
        m_ref[0] = jnp.broadcast_to(m_new, (H, D))
        l_ref[0] = jnp.broadcast_to(l_new := l_acc, (H, D))

    def kv_map(t):
        def f(i, j, loc, bt, lens):
            jj = j * CH + t
            page = bt[i, jj]
            lp = page - loc[0]
            ok = (jj < lens[i]) & (lp >= 0) & (lp < NP)
            return (jnp.where(ok, lp, 0), 0, 0, 0)
        return f

    def out_map(i, j, loc, bt, lens):
        return (i, 0, 0)

    grid_spec = pltpu.PrefetchScalarGridSpec(
        num_scalar_prefetch=3,
        grid=(B, NB // CH),
        in_specs=(
            [pl.BlockSpec((1, 1, H, D), lambda i, j, loc, bt, lens: (i, 0, 0, 0))]
            + [pl.BlockSpec((1, BS, H, D), kv_map(t)) for t in range(CH)]
            + [pl.BlockSpec((1, BS, H, D), kv_map(t)) for t in range(CH)]
        ),
        out_specs=[
            pl.BlockSpec((1, H, D), out_map),
            pl.BlockSpec((1, H, D), out_map),
            pl.BlockSpec((1, H, D), out_map),
        ],
    )
    return pl.pallas_call(
        body,
        grid_spec=grid_spec,
        out_shape=[jax.ShapeDtypeStruct((B, H, D), jnp.float32)] * 3,
        compiler_params=pltpu.CompilerParams(
            dimension_semantics=("arbitrary", "arbitrary"),
        ),
    )(loc, bt, lens, Q, *([K] * CH), *([V] * CH))


def _cross_x_combine(o, m, l):
    B, H, D = o.shape

    def body(o_ref, m_ref, l_ref, out_ref, ob, mb, lb, ssem, rsem):
        my_x = lax.axis_index("x")
        my_y = lax.axis_index("y")
        peer = (1 - my_x, my_y)
        rdmas = []
        for t, (src, dst) in enumerate(((o_ref, ob), (m_ref, mb), (l_ref, lb))):
            r = pltpu.make_async_remote_copy(
                src_ref=src,
                dst_ref=dst,
                send_sem=ssem.at[t],
                recv_sem=rsem.at[t],
                device_id=peer,
                device_id_type=pl.DeviceIdType.MESH,
            )
            r.start()
            rdmas.append(r)
        for r in rdmas:
            r.wait()
        m0 = m_ref[...]
        m1 = mb[...]
        mn = jnp.maximum(m0, m1)
        a0 = jnp.exp(m0 - mn)
        a1 = jnp.exp(m1 - mn)
        den = l_ref[...] * a0 + lb[...] * a1
        num = o_ref[...] * a0 + ob[...] * a1
        out_ref[...] = (num / den).reshape(B, 1, H, D)

    return pl.pallas_call(
        body,
        out_shape=jax.ShapeDtypeStruct((B, 1, H, D), jnp.float32),
        in_specs=[pl.BlockSpec(memory_space=pltpu.VMEM)] * 3,
        out_specs=pl.BlockSpec(memory_space=pltpu.VMEM),
        scratch_shapes=[
            pltpu.VMEM((B, H, D), jnp.float32),
            pltpu.VMEM((B, H, D), jnp.float32),
            pltpu.VMEM((B, H, D), jnp.float32),
            pltpu.SemaphoreType.DMA((3,)),
            pltpu.SemaphoreType.DMA((3,)),
        ],
    )(o, m, l)


def kernel(Q, K, V, bt, lens):
    NP = K.shape[0]
    loc = jnp.reshape(lax.axis_index("x").astype(jnp.int32) * NP, (1,))
    K16 = _cast_bf16(K)
    V16 = _cast_bf16(V)
    o, m, l = _local_paged_attn(loc, bt, lens, Q, K16, V16)
    return _cross_x_combine(o, m, l)


# device time: 414146 ns/iter; 25.9012x vs baseline; 17.2622x over previous
import jax
import jax.numpy as jnp
from jax import lax
from jax.experimental import pallas as pl
from jax.experimental.pallas import tpu as pltpu

_NEG = -1e30
_CH = 32


def _cast_transpose(x, yv):
    NP, BS, H, D = x.shape
    H2 = H // 2
    blk = 16

    def body(y_ref, x_ref, o_ref):
        o_ref[...] = x_ref[...].transpose(0, 2, 1, 3).astype(jnp.bfloat16)

    grid_spec = pltpu.PrefetchScalarGridSpec(
        num_scalar_prefetch=1,
        grid=(NP // blk,),
        in_specs=[pl.BlockSpec((blk, BS, H2, D), lambda i, y: (i, 0, y[0], 0))],
        out_specs=pl.BlockSpec((blk, H2, BS, D), lambda i, y: (i, 0, 0, 0)),
    )
    return pl.pallas_call(
        body,
        grid_spec=grid_spec,
        out_shape=jax.ShapeDtypeStruct((NP, H2, BS, D), jnp.bfloat16),
    )(yv, x)


def _local_paged_attn(loc, bt, lens, Q, Kt, Vt):
    B, _, H, D = Q.shape
    NP, H2, BS, _ = Kt.shape
    NB = bt.shape[1]
    CH = _CH
    NJ = NB // CH
    scale = D ** -0.5

    def body(loc_ref, bt_ref, lens_ref, q_ref, kt_ref, vt_ref,
             o_ref, m_ref, l_ref, kbuf, vbuf, ksem, vsem):
        i = pl.program_id(0)
        j = pl.program_id(1)
        slot = (i * NJ + j) % 2
        lo = loc_ref[0]

        def chunk_needed(bi, bj):
            return bj * CH < lens_ref[bi]

        def slot_ok(bi, bj, t):
            jj = bj * CH + t
            page = bt_ref[bi, jj]
            lp = page - lo
            return (jj < lens_ref[bi]) & (lp >= 0) & (lp < NP), lp

        def n_ok(bi, bj):
            n = jnp.int32(0)
            for t in range(CH):
                ok, _ = slot_ok(bi, bj, t)
                n = n + ok.astype(jnp.int32)
            return n

        def issue(bi, bj, sl):
            for t in range(CH):
                ok, lp = slot_ok(bi, bj, t)

                @pl.when(ok)
                def _():
                    pltpu.make_async_copy(
                        kt_ref.at[lp],
                        kbuf.at[sl, :, pl.ds(t * BS, BS), :],
                        ksem.at[sl],
                    ).start()
                    pltpu.make_async_copy(
                        vt_ref.at[lp],
                        vbuf.at[sl, :, pl.ds(t * BS, BS), :],
                        vsem.at[sl],
                    ).start()

        def wait(sl, n):
            def w(_, c):
                pltpu.make_async_copy(
                    kt_ref.at[0], kbuf.at[sl, :, pl.ds(0, BS), :], ksem.at[sl]
                ).wait()
                pltpu.make_async_copy(
                    vt_ref.at[0], vbuf.at[sl, :, pl.ds(0, BS), :], vsem.at[sl]
                ).wait()
                return c
            lax.fori_loop(0, n, w, 0)

        @pl.when((i == 0) & (j == 0))
        def _():
            vbuf[...] = jnp.zeros_like(vbuf)
            issue(0, 0, 0)

        nxt_i = jnp.where(j == NJ - 1, i + 1, i)
        nxt_j = jnp.where(j == NJ - 1, 0, j + 1)
        nxt_i_c = jnp.minimum(nxt_i, B - 1)

        @pl.when(((i < B - 1) | (j < NJ - 1)) & chunk_needed(nxt_i_c, nxt_j))
        def _():
            issue(nxt_i_c, nxt_j, 1 - slot)

        @pl.when(j == 0)
        def _():
            o_ref[...] = jnp.zeros_like(o_ref)
            m_ref[...] = jnp.full_like(m_ref, _NEG)
            l_ref[...] = jnp.zeros_like(l_ref)

        @pl.when(chunk_needed(i, j))
        def _():
            wait(slot, n_ok(i, j))
            q = q_ref[0, 0].astype(jnp.bfloat16)

            mask_parts = []
            for t in range(CH):
                ok, _ = slot_ok(i, j, t)
                mask_parts.append(jnp.full((1, BS), ok.astype(jnp.float32)))
            okvec = jnp.concatenate(mask_parts, axis=1) > 0.5

            kc = kbuf[slot]
            s = jnp.where(
                okvec,
                lax.dot_general(
                    q, kc, (((1,), (2,)), ((0,), (0,))),
                    preferred_element_type=jnp.float32,
                ) * scale,
                _NEG,
            )

            m_prev = m_ref[0, :, 0:1]
            l_prev = l_ref[0, :, 0:1]
            m_cur = jnp.max(s, axis=1, keepdims=True)
            m_new = jnp.maximum(m_prev, m_cur)
            alpha = jnp.exp(m_prev - m_new)
            p = jnp.exp(s - m_new)
            p = jnp.where(m_new > -1e29, p, 0.0)
            l_new = l_prev * alpha + jnp.sum(p, axis=1, keepdims=True)

            pv = lax.dot_general(
                p.astype(jnp.bfloat16), vbuf[slot],
                (((1,), (1,)), ((0,), (0,))),
                preferred_element_type=jnp.float32,
            )
            o_ref[0] = o_ref[0] * alpha + pv
            m_ref[0] = jnp.broadcast_to(m_new, (H2, D))
            l_ref[0] = jnp.broadcast_to(l_new, (H2, D))

    def out_map(i, j, loc, bt, lens):
        return (i, 0, 0)

    grid_spec = pltpu.PrefetchScalarGridSpec(
        num_scalar_prefetch=3,
        grid=(B, NJ),
        in_specs=[
            pl.BlockSpec((1, 1, H2, D),
                         lambda i, j, loc, bt, lens: (i, 0, loc[1], 0)),
            pl.BlockSpec(memory_space=pl.ANY),
            pl.BlockSpec(memory_space=pl.ANY),
        ],
        out_specs=[
            pl.BlockSpec((1, H2, D), out_map),
            pl.BlockSpec((1, H2, D), out_map),
            pl.BlockSpec((1, H2, D), out_map),
        ],
        scratch_shapes=[
            pltpu.VMEM((2, H2, CH * BS, D), jnp.bfloat16),
            pltpu.VMEM((2, H2, CH * BS, D), jnp.bfloat16),
            pltpu.SemaphoreType.DMA((2,)),
            pltpu.SemaphoreType.DMA((2,)),
        ],
    )
    return pl.pallas_call(
        body,
        grid_spec=grid_spec,
        out_shape=[jax.ShapeDtypeStruct((B, H2, D), jnp.float32)] * 3,
        compiler_params=pltpu.CompilerParams(
            dimension_semantics=("arbitrary", "arbitrary"),
        ),
    )(loc, bt, lens, Q, Kt, Vt)


def _combine(o, m, l):
    B, H2, D = o.shape
    H = 2 * H2

    def body(o_ref, m_ref, l_ref, out_ref, ob, mb, lb, fin, theirs, ssem, rsem):
        my_x = lax.axis_index("x")
        my_y = lax.axis_index("y")

        xpeer = (1 - my_x, my_y)
        rdmas = []
        for t, (src, dst) in enumerate(((o_ref, ob), (m_ref, mb), (l_ref, lb))):
            r = pltpu.make_async_remote_copy(
                src_ref=src,
                dst_ref=dst,
                send_sem=ssem.at[t],
                recv_sem=rsem.at[t],
                device_id=xpeer,
                device_id_type=pl.DeviceIdType.MESH,
            )
            r.start()
            rdmas.append(r)
        for r in rdmas:
            r.wait()
        m0 = m_ref[...]
        m1 = mb[...]
        mn = jnp.maximum(m0, m1)
        a0 = jnp.exp(m0 - mn)
        a1 = jnp.exp(m1 - mn)
        den = l_ref[...] * a0 + lb[...] * a1
        fin[...] = (o_ref[...] * a0 + ob[...] * a1) / den

        r = pltpu.make_async_remote_copy(
            src_ref=fin,
            dst_ref=theirs,
            send_sem=ssem.at[3],
            recv_sem=rsem.at[3],
            device_id=(my_x, 1 - my_y),
            device_id_type=pl.DeviceIdType.MESH,
        )
        r.start()
        r.wait()
        mine = fin[...]
        other = theirs[...]
        first = jnp.where(my_y == 0, mine, other)
        second = jnp.where(my_y == 0, other, mine)
        out_ref[...] = jnp.concatenate([first, second], axis=1).reshape(B, 1, H, D)

    return pl.pallas_call(
        body,
        out_shape=jax.ShapeDtypeStruct((B, 1, H, D), jnp.float32),
        in_specs=[pl.BlockSpec(memory_space=pltpu.VMEM)] * 3,
        out_specs=pl.BlockSpec(memory_space=pltpu.VMEM),
        scratch_shapes=[
            pltpu.VMEM((B, H2, D), jnp.float32),
            pltpu.VMEM((B, H2, D), jnp.float32),
            pltpu.VMEM((B, H2, D), jnp.float32),
            pltpu.VMEM((B, H2, D), jnp.float32),
            pltpu.VMEM((B, H2, D), jnp.float32),
            pltpu.SemaphoreType.DMA((4,)),
            pltpu.SemaphoreType.DMA((4,)),
        ],
    )(o, m, l)


def kernel(Q, K, V, bt, lens):
    NP = K.shape[0]
    xi = lax.axis_index("x").astype(jnp.int32)
    yi = lax.axis_index("y").astype(jnp.int32)
    loc = jnp.stack([xi * NP, yi])
    yv = jnp.reshape(yi, (1,))
    Kt = _cast_transpose(K, yv)
    Vt = _cast_transpose(V, yv)
    o, m, l = _local_paged_attn(loc, bt, lens, Q, Kt, Vt)
    return _combine(o, m, l)


# device time: 404210 ns/iter; 26.5379x vs baseline; 1.0246x over previous
import jax
import jax.numpy as jnp
from jax import lax
from jax.experimental import pallas as pl
from jax.experimental.pallas import tpu as pltpu

_NEG = -1e30
_CH = 64


def _cast_transpose(x, yv):
    NP, BS, H, D = x.shape
    H2 = H // 2
    blk = 16

    def body(y_ref, x_ref, o_ref):
        o_ref[...] = x_ref[...].transpose(0, 2, 1, 3).astype(jnp.bfloat16)

    grid_spec = pltpu.PrefetchScalarGridSpec(
        num_scalar_prefetch=1,
        grid=(NP // blk,),
        in_specs=[pl.BlockSpec((blk, BS, H2, D), lambda i, y: (i, 0, y[0], 0))],
        out_specs=pl.BlockSpec((blk, H2, BS, D), lambda i, y: (i, 0, 0, 0)),
    )
    return pl.pallas_call(
        body,
        grid_spec=grid_spec,
        out_shape=jax.ShapeDtypeStruct((NP, H2, BS, D), jnp.bfloat16),
    )(yv, x)


def _local_paged_attn(loc, bt, lens, Q, Kt, Vt):
    B, _, H, D = Q.shape
    NP, H2, BS, _ = Kt.shape
    NB = bt.shape[1]
    CH = _CH
    NJ = NB // CH
    scale = D ** -0.5

    def body(loc_ref, bt_ref, lens_ref, q_ref, kt_ref, vt_ref,
             o_ref, m_ref, l_ref, kbuf, vbuf, ksem, vsem):
        i = pl.program_id(0)
        j = pl.program_id(1)
        slot = (i * NJ + j) % 2
        lo = loc_ref[0]

        def chunk_needed(bi, bj):
            return bj * CH < lens_ref[bi]

        def slot_ok(bi, bj, t):
            jj = bj * CH + t
            page = bt_ref[bi, jj]
            lp = page - lo
            return (jj < lens_ref[bi]) & (lp >= 0) & (lp < NP), lp

        def n_ok(bi, bj):
            n = jnp.int32(0)
            for t in range(CH):
                ok, _ = slot_ok(bi, bj, t)
                n = n + ok.astype(jnp.int32)
            return n

        def issue(bi, bj, sl):
            for t in range(CH):
                ok, lp = slot_ok(bi, bj, t)

                @pl.when(ok)
                def _():
                    pltpu.make_async_copy(
                        kt_ref.at[lp],
                        kbuf.at[sl, :, pl.ds(t * BS, BS), :],
                        ksem.at[sl],
                    ).start()
                    pltpu.make_async_copy(
                        vt_ref.at[lp],
                        vbuf.at[sl, :, pl.ds(t * BS, BS), :],
                        vsem.at[sl],
                    ).start()

        def wait(sl, n):
            def w(_, c):
                pltpu.make_async_copy(
                    kt_ref.at[0], kbuf.at[sl, :, pl.ds(0, BS), :], ksem.at[sl]
                ).wait()
                pltpu.make_async_copy(
                    vt_ref.at[0], vbuf.at[sl, :, pl.ds(0, BS), :], vsem.at[sl]
                ).wait()
                return c
            lax.fori_loop(0, n, w, 0)

        @pl.when((i == 0) & (j == 0))
        def _():
            vbuf[...] = jnp.zeros_like(vbuf)
            issue(0, 0, 0)

        nxt_i = jnp.where(j == NJ - 1, i + 1, i)
        nxt_j = jnp.where(j == NJ - 1, 0, j + 1)
        nxt_i_c = jnp.minimum(nxt_i, B - 1)

        @pl.when(((i < B - 1) | (j < NJ - 1)) & chunk_needed(nxt_i_c, nxt_j))
        def _():
            issue(nxt_i_c, nxt_j, 1 - slot)

        @pl.when(j == 0)
        def _():
            o_ref[...] = jnp.zeros_like(o_ref)
            m_ref[...] = jnp.full_like(m_ref, _NEG)
            l_ref[...] = jnp.zeros_like(l_ref)

        def compute(ksl, vsl):
            q = q_ref[0, 0].astype(jnp.bfloat16)

            mask_parts = []
            for t in range(CH):
                ok, _ = slot_ok(i, j, t)
                mask_parts.append(jnp.full((1, BS), ok.astype(jnp.float32)))
            okvec = jnp.concatenate(mask_parts, axis=1) > 0.5

            s = jnp.where(
                okvec,
                lax.dot_general(
                    q, ksl[...], (((1,), (2,)), ((0,), (0,))),
                    preferred_element_type=jnp.float32,
                ) * scale,
                _NEG,
            )

            m_prev = m_ref[0, :, 0:1]
            l_prev = l_ref[0, :, 0:1]
            m_cur = jnp.max(s, axis=1, keepdims=True)
            m_new = jnp.maximum(m_prev, m_cur)
            alpha = jnp.exp(m_prev - m_new)
            p = jnp.exp(s - m_new)
            p = jnp.where(m_new > -1e29, p, 0.0)
            l_new = l_prev * alpha + jnp.sum(p, axis=1, keepdims=True)

            pv = lax.dot_general(
                p.astype(jnp.bfloat16), vsl[...],
                (((1,), (1,)), ((0,), (0,))),
                preferred_element_type=jnp.float32,
            )
            o_ref[0] = o_ref[0] * alpha + pv
            m_ref[0] = jnp.broadcast_to(m_new, (H2, D))
            l_ref[0] = jnp.broadcast_to(l_new, (H2, D))

        @pl.when(chunk_needed(i, j))
        def _():
            wait(slot, n_ok(i, j))

            @pl.when(slot == 0)
            def _():
                compute(kbuf.at[0], vbuf.at[0])

            @pl.when(slot == 1)
            def _():
                compute(kbuf.at[1], vbuf.at[1])

    def out_map(i, j, loc, bt, lens):
        return (i, 0, 0)

    grid_spec = pltpu.PrefetchScalarGridSpec(
        num_scalar_prefetch=3,
        grid=(B, NJ),
        in_specs=[
            pl.BlockSpec((1, 1, H2, D),
                         lambda i, j, loc, bt, lens: (i, 0, loc[1], 0)),
            pl.BlockSpec(memory_space=pl.ANY),
            pl.BlockSpec(memory_space=pl.ANY),
        ],
        out_specs=[
            pl.BlockSpec((1, H2, D), out_map),
            pl.BlockSpec((1, H2, D), out_map),
            pl.BlockSpec((1, H2, D), out_map),
        ],
        scratch_shapes=[
            pltpu.VMEM((2, H2, CH * BS, D), jnp.bfloat16),
            pltpu.VMEM((2, H2, CH * BS, D), jnp.bfloat16),
            pltpu.SemaphoreType.DMA((2,)),
            pltpu.SemaphoreType.DMA((2,)),
        ],
    )
    return pl.pallas_call(
        body,
        grid_spec=grid_spec,
        out_shape=[jax.ShapeDtypeStruct((B, H2, D), jnp.float32)] * 3,
        compiler_params=pltpu.CompilerParams(
            dimension_semantics=("arbitrary", "arbitrary"),
        ),
    )(loc, bt, lens, Q, Kt, Vt)


def _combine(o, m, l):
    B, H2, D = o.shape
    H = 2 * H2

    def body(o_ref, m_ref, l_ref, out_ref, ob, mb, lb, fin, theirs, ssem, rsem):
        my_x = lax.axis_index("x")
        my_y = lax.axis_index("y")

        xpeer = (1 - my_x, my_y)
        rdmas = []
        for t, (src, dst) in enumerate(((o_ref, ob), (m_ref, mb), (l_ref, lb))):
            r = pltpu.make_async_remote_copy(
                src_ref=src,
                dst_ref=dst,
                send_sem=ssem.at[t],
                recv_sem=rsem.at[t],
                device_id=xpeer,
                device_id_type=pl.DeviceIdType.MESH,
            )
            r.start()
            rdmas.append(r)
        for r in rdmas:
            r.wait()
        m0 = m_ref[...]
        m1 = mb[...]
        mn = jnp.maximum(m0, m1)
        a0 = jnp.exp(m0 - mn)
        a1 = jnp.exp(m1 - mn)
        den = l_ref[...] * a0 + lb[...] * a1
        fin[...] = (o_ref[...] * a0 + ob[...] * a1) / den

        r = pltpu.make_async_remote_copy(
            src_ref=fin,
            dst_ref=theirs,
            send_sem=ssem.at[3],
            recv_sem=rsem.at[3],
            device_id=(my_x, 1 - my_y),
            device_id_type=pl.DeviceIdType.MESH,
        )
        r.start()
        r.wait()
        mine = fin[...]
        other = theirs[...]
        first = jnp.where(my_y == 0, mine, other)
        second = jnp.where(my_y == 0, other, mine)
        out_ref[...] = jnp.concatenate([first, second], axis=1).reshape(B, 1, H, D)

    return pl.pallas_call(
        body,
        out_shape=jax.ShapeDtypeStruct((B, 1, H, D), jnp.float32),
        in_specs=[pl.BlockSpec(memory_space=pltpu.VMEM)] * 3,
        out_specs=pl.BlockSpec(memory_space=pltpu.VMEM),
        scratch_shapes=[
            pltpu.VMEM((B, H2, D), jnp.float32),
            pltpu.VMEM((B, H2, D), jnp.float32),
            pltpu.VMEM((B, H2, D), jnp.float32),
            pltpu.VMEM((B, H2, D), jnp.float32),
            pltpu.VMEM((B, H2, D), jnp.float32),
            pltpu.SemaphoreType.DMA((4,)),
            pltpu.SemaphoreType.DMA((4,)),
        ],
    )(o, m, l)


def kernel(Q, K, V, bt, lens):
    NP = K.shape[0]
    xi = lax.axis_index("x").astype(jnp.int32)
    yi = lax.axis_index("y").astype(jnp.int32)
    loc = jnp.stack([xi * NP, yi])
    yv = jnp.reshape(yi, (1,))
    Kt = _cast_transpose(K, yv)
    Vt = _cast_transpose(V, yv)
    o, m, l = _local_paged_attn(loc, bt, lens, Q, Kt, Vt)
    return _combine(o, m, l)
